# baseline (speedup 1.0000x reference)
import functools

import jax
import jax.numpy as jnp
from jax import lax
from jax.experimental import pallas as pl
from jax.experimental.pallas import tpu as pltpu
from jax.experimental.pallas import tpu_sc as plsc

B = 16384
D = 64
W = 128
N = 1000000
NC = 2
NS = 16
NW = NC * NS
CHUNK = 128
BPW = B // NW
NCHUNK = BPW // CHUNK
NFULL = N // CHUNK
TAIL = N - NFULL * CHUNK
TAIL_BASE = NFULL * CHUNK

_mesh = plsc.VectorSubcoreMesh(core_axis_name="c", subcore_axis_name="s")


@functools.partial(
    pl.kernel,
    mesh=_mesh,
    compiler_params=pltpu.CompilerParams(needs_layout_passes=False),
    out_type=(
        jax.ShapeDtypeStruct((N, W), jnp.float32),
        jax.ShapeDtypeStruct((N, W), jnp.float32),
    ),
    scratch_types=[
        pltpu.VMEM((2, D, CHUNK), jnp.float32),
        pltpu.VMEM((2, CHUNK, W), jnp.float32),
        pltpu.SemaphoreType.DMA,
        pltpu.SemaphoreType.DMA,
    ],
)
def _convert(ent_t, rel_t, out_e, out_r, buf, stage, sem_in, sem_out):
    wid = lax.axis_index("s") * NC + lax.axis_index("c")
    lane = lax.iota(jnp.int32, 16)
    rows = [lane + 16 * rv for rv in range(8)]

    def in_wait():
        pltpu.make_async_copy(ent_t.at[:, pl.ds(0, CHUNK)], buf.at[0],
                              sem_in).wait()

    def out_wait():
        pltpu.make_async_copy(out_e.at[pl.ds(0, CHUNK)], stage.at[0],
                              sem_out).wait()

    def run_table(src, dst):
        nblk = (NFULL - wid + NW - 1) // NW

        def base_of(i):
            return (wid + i * NW) * CHUNK

        @pl.when(nblk > 0)
        def _():
            pltpu.async_copy(src.at[:, pl.ds(base_of(0), CHUNK)],
                             buf.at[0], sem_in)

        def body(i, carry):
            sl = i % 2
            in_wait()

            @pl.when(i + 1 < nblk)
            def _():
                pltpu.async_copy(src.at[:, pl.ds(base_of(i + 1), CHUNK)],
                                 buf.at[sl ^ 1], sem_in)

            @pl.when(i >= 2)
            def _():
                out_wait()

            def col_body(c, carry2):
                col = jnp.full((16,), c, jnp.int32)
                for rv in range(8):
                    v = buf[sl, c, pl.ds(rv * 16, 16)]
                    plsc.store_scatter(stage.at[sl], [rows[rv], col], v)
                return carry2

            lax.fori_loop(0, D, col_body, 0)
            pltpu.async_copy(stage.at[sl], dst.at[pl.ds(base_of(i), CHUNK)],
                             sem_out)
            return carry

        lax.fori_loop(0, nblk, body, 0)
        @pl.when(nblk >= 2)
        def _():
            out_wait()

        @pl.when(nblk >= 1)
        def _():
            out_wait()

    run_table(ent_t, out_e)
    run_table(rel_t, out_r)


@functools.partial(
    pl.kernel,
    mesh=_mesh,
    out_type=(
        jax.ShapeDtypeStruct((NW, NCHUNK, CHUNK, W), jnp.float32),
        jax.ShapeDtypeStruct((NW, NCHUNK, CHUNK, W), jnp.float32),
        jax.ShapeDtypeStruct((NW, NCHUNK, CHUNK, W), jnp.float32),
    ),
    scratch_types=[
        pltpu.VMEM((NCHUNK, CHUNK), jnp.int32),
        pltpu.VMEM((NCHUNK, CHUNK), jnp.int32),
        pltpu.VMEM((NCHUNK, CHUNK), jnp.int32),
        pltpu.VMEM((NCHUNK, CHUNK, W), jnp.float32),
        pltpu.SemaphoreType.DMA,
        pltpu.SemaphoreType.DMA,
    ],
)
def _gather3(h_idx, r_idx, t_idx, ent, rel, out_h, out_r, out_t,
             hv, rv, tv, rows, sem_i, sem_g):
    wid = lax.axis_index("s") * NC + lax.axis_index("c")
    idx_copies = [
        pltpu.async_copy(h_idx.at[wid], hv, sem_i),
        pltpu.async_copy(r_idx.at[wid], rv, sem_i),
        pltpu.async_copy(t_idx.at[wid], tv, sem_i),
    ]
    for c in idx_copies:
        c.wait()
    for table, idx_v, out in ((ent, hv, out_h), (rel, rv, out_r),
                              (ent, tv, out_t)):
        gathers = [
            pltpu.async_copy(table.at[idx_v.at[j]], rows.at[j], sem_g)
            for j in range(NCHUNK)
        ]
        for c in gathers:
            c.wait()
        pltpu.sync_copy(rows, out.at[wid])


def _tail_fix(vals, idx, table):
    tail = jnp.take(table[TAIL_BASE:], jnp.clip(idx - TAIL_BASE, 0, TAIL - 1),
                    axis=0)
    return jnp.where((idx >= TAIL_BASE)[:, None, None], tail[:, None, :], vals)


def kernel(sample, entity_embedding, relation_embedding):
    pe, pr = _convert(entity_embedding.T, relation_embedding.T)
    idx = sample.T.reshape(3, NW, NCHUNK, CHUNK)
    h, r, t = _gather3(idx[0], idx[1], idx[2], pe, pr)
    h = h.reshape(B, W)[:, None, :D]
    r = r.reshape(B, W)[:, None, :D]
    t = t.reshape(B, W)[:, None, :D]
    i0, i1, i2 = sample[:, 0], sample[:, 1], sample[:, 2]
    h = _tail_fix(h, i0, entity_embedding)
    r = _tail_fix(r, i1, relation_embedding)
    t = _tail_fix(t, i2, entity_embedding)
    return (h, r, t)


# speedup vs baseline: 2.4759x; 2.4759x over previous
import functools

import jax
import jax.numpy as jnp
from jax import lax
from jax.experimental import pallas as pl
from jax.experimental.pallas import tpu as pltpu
from jax.experimental.pallas import tpu_sc as plsc

B = 16384
D = 64
W = 128
N = 1000000
NC = 2
NS = 16
NW = NC * NS
CHUNK = 128
BPW = B // NW
NCHUNK = BPW // CHUNK
NFULL = N // CHUNK
TAIL = N - NFULL * CHUNK
TAIL_BASE = NFULL * CHUNK

_mesh = plsc.VectorSubcoreMesh(core_axis_name="c", subcore_axis_name="s")


@functools.partial(
    pl.kernel,
    mesh=_mesh,
    out_type=(
        jax.ShapeDtypeStruct((NW, NCHUNK, CHUNK, W), jnp.float32),
        jax.ShapeDtypeStruct((NW, NCHUNK, CHUNK, W), jnp.float32),
        jax.ShapeDtypeStruct((NW, NCHUNK, CHUNK, W), jnp.float32),
    ),
    scratch_types=[
        pltpu.VMEM((NCHUNK, CHUNK), jnp.int32),
        pltpu.VMEM((NCHUNK, CHUNK), jnp.int32),
        pltpu.VMEM((NCHUNK, CHUNK), jnp.int32),
        pltpu.VMEM((NCHUNK, CHUNK, W), jnp.float32),
        pltpu.SemaphoreType.DMA,
        pltpu.SemaphoreType.DMA,
    ],
)
def _gather3(h_idx, r_idx, t_idx, ent, rel, out_h, out_r, out_t,
             hv, rv, tv, rows, sem_i, sem_g):
    wid = lax.axis_index("s") * NC + lax.axis_index("c")
    idx_copies = [
        pltpu.async_copy(h_idx.at[wid], hv, sem_i),
        pltpu.async_copy(r_idx.at[wid], rv, sem_i),
        pltpu.async_copy(t_idx.at[wid], tv, sem_i),
    ]
    for c in idx_copies:
        c.wait()
    for table, idx_v, out in ((ent, hv, out_h), (rel, rv, out_r),
                              (ent, tv, out_t)):
        gathers = [
            pltpu.async_copy(table.at[idx_v.at[j]], rows.at[j], sem_g)
            for j in range(NCHUNK)
        ]
        for c in gathers:
            c.wait()
        pltpu.sync_copy(rows, out.at[wid])


def kernel(sample, entity_embedding, relation_embedding):
    pe = jnp.pad(entity_embedding, ((0, 0), (0, W - D)))
    pr = jnp.pad(relation_embedding, ((0, 0), (0, W - D)))
    idx = sample.T.reshape(3, NW, NCHUNK, CHUNK)
    h, r, t = _gather3(idx[0], idx[1], idx[2], pe, pr)
    h = h.reshape(B, W)
    r = r.reshape(B, W)
    t = t.reshape(B, W)
    return (h[:, None, :D], r[:, None, :D], t[:, None, :D])
